# baseline (speedup 1.0000x reference)
import functools
import math

import jax
import jax.numpy as jnp
from jax import lax
from jax.experimental import pallas as pl
from jax.experimental.pallas import tpu as pltpu
from jax.experimental.pallas import tpu_sc as plsc

_UNITS = 1000000
_NEG = 8192
_BATCH = 4096
_DIM = 64

_SB = 2048
_NS = _NEG // _SB
_LOG_UNITS1 = math.log(_UNITS + 1.0)
_LOG_NEG = math.log(float(_NEG))


def _sc_gather(table_t, bias, idx):
    ntot = idx.shape[0]
    info = plsc.get_sparse_core_info()
    nw = info.num_cores * info.num_subcores
    bpw = ntot // nw
    assert ntot % nw == 0 and bpw % 16 == 0

    @functools.partial(
        pl.kernel,
        mesh=plsc.VectorSubcoreMesh(core_axis_name="c", subcore_axis_name="s"),
        compiler_params=pltpu.CompilerParams(needs_layout_passes=False),
        out_type=(
            jax.ShapeDtypeStruct((ntot * _DIM,), jnp.float32),
            jax.ShapeDtypeStruct((ntot,), jnp.float32),
        ),
        scratch_types=[
            pltpu.VMEM((bpw,), jnp.int32),
            pltpu.VMEM((4 * _DIM, 128), jnp.float32),
            pltpu.VMEM((4 * _DIM, 128), jnp.float32),
            pltpu.VMEM((bpw * _DIM,), jnp.float32),
            pltpu.VMEM((bpw,), jnp.float32),
            pltpu.SemaphoreType.DMA,
            pltpu.SemaphoreType.DMA,
            pltpu.SemaphoreType.DMA,
        ],
    )
    def k(table_hbm, bias_hbm, idx_hbm, rows_out, brows_out,
          idx_v, buf_a, buf_b, rows_v, b_v, sem_a, sem_b, sem_bias):
        wid = lax.axis_index("s") * info.num_cores + lax.axis_index("c")
        base = wid * bpw
        pltpu.sync_copy(idx_hbm.at[pl.ds(base, bpw)], idx_v)

        cb = pltpu.async_copy(bias_hbm.at[idx_v], b_v, sem_bias)

        lanes = lax.iota(jnp.int32, 16)
        zeros16 = jnp.zeros((16,), jnp.int32)

        def issue(vec, lane0, buf, sem):
            for t in range(4):
                ij = vec[lane0 + t]
                blk = pl.multiple_of((ij // 128) * 128, 128)
                pltpu.async_copy(table_hbm.at[:, pl.ds(blk, 128)],
                                 buf.at[pl.ds(t * _DIM, _DIM), :], sem)

        def extract(vec, lane0, jbase, buf, sem):
            for t in range(4):
                pltpu.make_async_copy(
                    table_hbm.at[:, pl.ds(0, 128)],
                    buf.at[pl.ds(t * _DIM, _DIM), :], sem).wait()
            for t in range(4):
                ij = vec[lane0 + t]
                offv = zeros16 + (ij - (ij // 128) * 128)
                jv = zeros16 + (jbase + t)
                for q in range(4):
                    dv = lanes + (q * 16)
                    v = plsc.load_gather(buf, [dv + t * _DIM, offv])
                    plsc.store_scatter(rows_v, [jv * _DIM + dv], v)

        def body(g, carry):
            vec = idx_v[pl.ds(pl.multiple_of(g * 16, 16), 16)]
            jb = g * 16
            issue(vec, 0, buf_a, sem_a)
            issue(vec, 4, buf_b, sem_b)
            extract(vec, 0, jb, buf_a, sem_a)
            issue(vec, 8, buf_a, sem_a)
            extract(vec, 4, jb + 4, buf_b, sem_b)
            issue(vec, 12, buf_b, sem_b)
            extract(vec, 8, jb + 8, buf_a, sem_a)
            extract(vec, 12, jb + 12, buf_b, sem_b)
            return carry
        lax.fori_loop(0, bpw // 16, body, 0)

        cb.wait()
        pltpu.sync_copy(rows_v, rows_out.at[pl.ds(base * _DIM, bpw * _DIM)])
        pltpu.sync_copy(b_v, brows_out.at[pl.ds(base, bpw)])

    return k(table_t, bias, idx)


def _neg_log_expected(ids_f32):
    p = (jnp.log(ids_f32 + 2.0) - jnp.log(ids_f32 + 1.0)) / _LOG_UNITS1
    return _LOG_NEG + jnp.log(p)


def _tc_body(tgt_ref, smp_ref, logits_ref, true_w_ref, samp_w_ref,
             true_b_ref, samp_b_ref, out_ref, m_sc, l_sc, tl_sc):
    s = pl.program_id(0)
    logits = logits_ref[...]

    @pl.when(s == 0)
    def _init():
        tw = true_w_ref[...]
        tb = true_b_ref[...]
        tgt_f = tgt_ref[...].astype(jnp.float32)
        tl = (jnp.sum(logits * tw, axis=1, keepdims=True)
              + tb - _neg_log_expected(tgt_f))
        tl_sc[...] = tl
        m_sc[...] = tl
        l_sc[...] = jnp.ones_like(tl)

    w = samp_w_ref[...]
    sb = samp_b_ref[...]
    smp = smp_ref[...]
    adj = sb - _neg_log_expected(smp.astype(jnp.float32))
    x = lax.dot_general(logits, w, (((1,), (1,)), ((), ())),
                        preferred_element_type=jnp.float32)
    x = x + adj
    hit = (tgt_ref[...] == smp).astype(jnp.float32)
    x = x - hit * 1e9

    m_prev = m_sc[...]
    l_prev = l_sc[...]
    m_new = jnp.maximum(m_prev, jnp.max(x, axis=1, keepdims=True))
    l_new = (l_prev * jnp.exp(m_prev - m_new)
             + jnp.sum(jnp.exp(x - m_new), axis=1, keepdims=True))
    m_sc[...] = m_new
    l_sc[...] = l_new

    @pl.when(s == _NS - 1)
    def _fin():
        per_ex = m_sc[...] + jnp.log(l_sc[...]) - tl_sc[...]
        out_ref[...] = jnp.sum(per_ex, axis=0, keepdims=True) / _BATCH


def _tc_loss(tgt_col, smp_row, logits, rows, true_b_col, samp_b_row):
    b = logits.shape[0]
    sampled_block0 = b // _SB
    grid_spec = pltpu.PrefetchScalarGridSpec(
        num_scalar_prefetch=0,
        grid=(_NS,),
        in_specs=[
            pl.BlockSpec((b, 1), lambda s: (0, 0)),
            pl.BlockSpec((1, _SB), lambda s: (0, s)),
            pl.BlockSpec((b, _DIM), lambda s: (0, 0)),
            pl.BlockSpec((b, _DIM), lambda s: (0, 0)),
            pl.BlockSpec((_SB, _DIM),
                         lambda s: (sampled_block0 + s, 0)),
            pl.BlockSpec((b, 1), lambda s: (0, 0)),
            pl.BlockSpec((1, _SB), lambda s: (0, s)),
        ],
        out_specs=pl.BlockSpec((1, 1), lambda s: (0, 0)),
        scratch_shapes=[
            pltpu.VMEM((b, 1), jnp.float32),
            pltpu.VMEM((b, 1), jnp.float32),
            pltpu.VMEM((b, 1), jnp.float32),
        ],
    )
    loss = pl.pallas_call(
        _tc_body,
        grid_spec=grid_spec,
        out_shape=jax.ShapeDtypeStruct((1, 1), jnp.float32),
        compiler_params=pltpu.CompilerParams(
            dimension_semantics=("arbitrary",),
        ),
    )(tgt_col, smp_row, logits, rows, rows, true_b_col, samp_b_row)
    return loss[0, 0]


def kernel(logits, targets, kernel, bias, sampled):
    idx = jnp.concatenate([targets, sampled])
    table_t = kernel.T
    rows_flat, brows = _sc_gather(table_t, bias, idx)
    rows = rows_flat.reshape(_BATCH + _NEG, _DIM)
    true_b = brows[:_BATCH].reshape(_BATCH, 1)
    samp_b = brows[_BATCH:].reshape(1, _NEG)
    tgt_col = targets.reshape(_BATCH, 1)
    smp_row = sampled.reshape(1, _NEG)
    return _tc_loss(tgt_col, smp_row, logits.reshape(-1, _DIM),
                    rows, true_b, samp_b)


# speedup vs baseline: 1.1213x; 1.1213x over previous
import functools
import math

import jax
import jax.numpy as jnp
from jax import lax
from jax.experimental import pallas as pl
from jax.experimental.pallas import tpu as pltpu
from jax.experimental.pallas import tpu_sc as plsc

_UNITS = 1000000
_NEG = 8192
_BATCH = 4096
_DIM = 64

_SB = 2048
_NS = _NEG // _SB
_LOG_UNITS1 = math.log(_UNITS + 1.0)
_LOG_NEG = math.log(float(_NEG))


def _sc_gather(table, bias, targets, sampled):
    nb = targets.shape[0]
    ns = sampled.shape[0]
    info = plsc.get_sparse_core_info()
    nw = info.num_cores * info.num_subcores
    bpt = nb // nw
    bps = ns // nw
    assert nb % nw == 0 and ns % nw == 0 and bpt % 16 == 0 and bps % 16 == 0

    @functools.partial(
        pl.kernel,
        mesh=plsc.VectorSubcoreMesh(core_axis_name="c", subcore_axis_name="s"),
        out_type=(
            jax.ShapeDtypeStruct((nb + ns, _DIM), jnp.float32),
            jax.ShapeDtypeStruct((nb,), jnp.float32),
            jax.ShapeDtypeStruct((ns,), jnp.float32),
        ),
        scratch_types=[
            pltpu.VMEM((bpt,), jnp.int32),
            pltpu.VMEM((bps,), jnp.int32),
            pltpu.VMEM((bpt + bps, _DIM), jnp.float32),
            pltpu.VMEM((bpt,), jnp.float32),
            pltpu.VMEM((bps,), jnp.float32),
            pltpu.SemaphoreType.DMA,
            pltpu.SemaphoreType.DMA,
            pltpu.SemaphoreType.DMA,
        ],
    )
    def k(table_hbm, bias_hbm, tgt_hbm, smp_hbm,
          rows_out, bt_out, bs_out,
          it_v, is_v, rows_v, bt_v, bs_v, sem1, sem2, sem3):
        wid = lax.axis_index("s") * info.num_cores + lax.axis_index("c")
        tbase = wid * bpt
        sbase = wid * bps
        pltpu.sync_copy(tgt_hbm.at[pl.ds(tbase, bpt)], it_v)
        pltpu.sync_copy(smp_hbm.at[pl.ds(sbase, bps)], is_v)

        cbt = pltpu.async_copy(bias_hbm.at[it_v], bt_v, sem2)
        cbs = pltpu.async_copy(bias_hbm.at[is_v], bs_v, sem3)

        def issue_t(g, carry):
            vec = it_v[pl.ds(pl.multiple_of(g * 16, 16), 16)]
            for kk in range(16):
                ij = vec[kk]
                pltpu.async_copy(table_hbm.at[pl.ds(ij, 1), :],
                                 rows_v.at[pl.ds(g * 16 + kk, 1), :], sem1)
            return carry
        lax.fori_loop(0, bpt // 16, issue_t, 0)

        def issue_s(g, carry):
            vec = is_v[pl.ds(pl.multiple_of(g * 16, 16), 16)]
            for kk in range(16):
                ij = vec[kk]
                pltpu.async_copy(
                    table_hbm.at[pl.ds(ij, 1), :],
                    rows_v.at[pl.ds(bpt + g * 16 + kk, 1), :], sem1)
            return carry
        lax.fori_loop(0, bps // 16, issue_s, 0)

        pltpu.make_async_copy(
            table_hbm.at[pl.ds(0, bpt + bps), :], rows_v, sem1).wait()
        cbt.wait()
        cbs.wait()
        pltpu.sync_copy(rows_v.at[pl.ds(0, bpt)],
                        rows_out.at[pl.ds(tbase, bpt)])
        pltpu.sync_copy(rows_v.at[pl.ds(bpt, bps)],
                        rows_out.at[pl.ds(nb + sbase, bps)])
        pltpu.sync_copy(bt_v, bt_out.at[pl.ds(tbase, bpt)])
        pltpu.sync_copy(bs_v, bs_out.at[pl.ds(sbase, bps)])

    return k(table, bias, targets, sampled)


def _neg_log_expected(ids_f32):
    p = (jnp.log(ids_f32 + 2.0) - jnp.log(ids_f32 + 1.0)) / _LOG_UNITS1
    return _LOG_NEG + jnp.log(p)


def _tc_body(tgt_ref, smp_ref, logits_ref, true_w_ref, samp_w_ref,
             true_b_ref, samp_b_ref, out_ref, m_sc, l_sc, tl_sc):
    s = pl.program_id(0)
    logits = logits_ref[...]

    @pl.when(s == 0)
    def _init():
        tw = true_w_ref[...]
        tb = true_b_ref[...]
        tgt_f = tgt_ref[...].astype(jnp.float32)
        tl = (jnp.sum(logits * tw, axis=1, keepdims=True)
              + tb - _neg_log_expected(tgt_f))
        tl_sc[...] = tl
        m_sc[...] = tl
        l_sc[...] = jnp.ones_like(tl)

    w = samp_w_ref[...]
    sb = samp_b_ref[...]
    smp = smp_ref[...]
    adj = sb - _neg_log_expected(smp.astype(jnp.float32))
    x = lax.dot_general(logits, w, (((1,), (1,)), ((), ())),
                        preferred_element_type=jnp.float32)
    x = x + adj
    hit = (tgt_ref[...] == smp).astype(jnp.float32)
    x = x - hit * 1e9

    m_prev = m_sc[...]
    l_prev = l_sc[...]
    m_new = jnp.maximum(m_prev, jnp.max(x, axis=1, keepdims=True))
    l_new = (l_prev * jnp.exp(m_prev - m_new)
             + jnp.sum(jnp.exp(x - m_new), axis=1, keepdims=True))
    m_sc[...] = m_new
    l_sc[...] = l_new

    @pl.when(s == _NS - 1)
    def _fin():
        per_ex = m_sc[...] + jnp.log(l_sc[...]) - tl_sc[...]
        out_ref[...] = jnp.sum(per_ex, axis=0, keepdims=True) / _BATCH


def _tc_loss(tgt_col, smp_row, logits, rows, true_b_col, samp_b_row):
    b = logits.shape[0]
    sampled_block0 = b // _SB
    grid_spec = pltpu.PrefetchScalarGridSpec(
        num_scalar_prefetch=0,
        grid=(_NS,),
        in_specs=[
            pl.BlockSpec((b, 1), lambda s: (0, 0)),
            pl.BlockSpec((1, _SB), lambda s: (0, s)),
            pl.BlockSpec((b, _DIM), lambda s: (0, 0)),
            pl.BlockSpec((b, _DIM), lambda s: (0, 0)),
            pl.BlockSpec((_SB, _DIM),
                         lambda s: (sampled_block0 + s, 0)),
            pl.BlockSpec((b, 1), lambda s: (0, 0)),
            pl.BlockSpec((1, _SB), lambda s: (0, s)),
        ],
        out_specs=pl.BlockSpec((1, 1), lambda s: (0, 0)),
        scratch_shapes=[
            pltpu.VMEM((b, 1), jnp.float32),
            pltpu.VMEM((b, 1), jnp.float32),
            pltpu.VMEM((b, 1), jnp.float32),
        ],
    )
    loss = pl.pallas_call(
        _tc_body,
        grid_spec=grid_spec,
        out_shape=jax.ShapeDtypeStruct((1, 1), jnp.float32),
        compiler_params=pltpu.CompilerParams(
            dimension_semantics=("arbitrary",),
        ),
    )(tgt_col, smp_row, logits, rows, rows, true_b_col, samp_b_row)
    return loss[0, 0]


def kernel(logits, targets, kernel, bias, sampled):
    rows, bt, bs = _sc_gather(kernel, bias, targets, sampled)
    true_b = bt.reshape(_BATCH, 1)
    samp_b = bs.reshape(1, _NEG)
    tgt_col = targets.reshape(_BATCH, 1)
    smp_row = sampled.reshape(1, _NEG)
    return _tc_loss(tgt_col, smp_row, logits.reshape(-1, _DIM),
                    rows, true_b, samp_b)
